# baseline (speedup 1.0000x reference)
import functools

import jax
import jax.numpy as jnp
from jax import lax
from jax.experimental import pallas as pl
from jax.experimental.pallas import tpu as pltpu
from jax.experimental.pallas import tpu_sc as plsc

B, S, D = 8, 2048, 384
MAXM = 14336
NTILES = 32
TPB = NTILES // B
CHUNK = 64
SPAN = 32
NCHUNK = MAXM // (CHUNK * TPB)
SV = S // 16
CV = CHUNK // 16
DV = D // 16
NBLK = 4
NOUT = 2
STAGE_LAG = 2


@functools.partial(
    pl.kernel,
    out_type=(
        jax.ShapeDtypeStruct((B * MAXM, D), jnp.float32),
        jax.ShapeDtypeStruct((B, 16), jnp.int32),
    ),
    mesh=plsc.VectorSubcoreMesh(core_axis_name="c", subcore_axis_name="s"),
    compiler_params=pltpu.CompilerParams(needs_layout_passes=False),
    scratch_types=[
        pltpu.VMEM((S,), jnp.int32),
        pltpu.VMEM((S,), jnp.int32),
        pltpu.VMEM((NBLK, CHUNK), jnp.int32),
        pltpu.VMEM((NBLK, 16), jnp.int32),
        pltpu.VMEM((SPAN, D), jnp.float32),
        pltpu.VMEM((SPAN, D), jnp.float32),
        pltpu.VMEM((SPAN, D), jnp.float32),
        pltpu.VMEM((SPAN, D), jnp.float32),
        pltpu.VMEM((CHUNK, D), jnp.float32),
        pltpu.VMEM((CHUNK, D), jnp.float32),
        pltpu.VMEM((CHUNK, D), jnp.float32),
        pltpu.VMEM((16,), jnp.int32),
        pltpu.SemaphoreType.DMA,
        pltpu.SemaphoreType.DMA,
        pltpu.SemaphoreType.DMA,
        pltpu.SemaphoreType.DMA,
        pltpu.SemaphoreType.DMA,
        pltpu.SemaphoreType.DMA,
        pltpu.SemaphoreType.DMA,
    ],
)
def _expand(x_hbm, dur_hbm, out_hbm, len_hbm,
            dur_ref, csum_ref, idx4, meta4, blk0, blk1, blk2, blk3,
            buf0, buf1, zbuf, lens_v,
            sem, gsem0, gsem1, gsem2, gsem3, ssem0, ssem1):
    cid = lax.axis_index("c")
    sid = lax.axis_index("s")
    wid = cid * 16 + sid
    b = wid // TPB
    q = wid % TPB
    src_base = b * S

    blks = (blk0, blk1, blk2, blk3)
    bufs = (buf0, buf1)
    gsems = (gsem0, gsem1, gsem2, gsem3)
    ssems = (ssem0, ssem1)

    iota16 = lax.broadcasted_iota(jnp.int32, (16,), 0)
    zv = jnp.zeros((16,), jnp.float32)

    gather_dnums = lax.GatherDimensionNumbers(
        offset_dims=(), collapsed_slice_dims=(0,), start_index_map=(0,))

    def lane_permute(v, idx):
        return lax.gather(v, idx[:, None], gather_dnums, slice_sizes=(1,),
                          mode=lax.GatherScatterMode.PROMISE_IN_BOUNDS)

    def lane_cumsum(v):
        s = v
        for k in (1, 2, 4, 8):
            sh = lane_permute(s, jnp.maximum(iota16 - k, 0))
            s = s + jnp.where(iota16 >= k, sh, 0)
        return s

    def lane_max(v):
        m = v
        for k in (1, 2, 4, 8):
            m = jnp.maximum(m, lane_permute(m, (iota16 + k) & 15))
        return m[0]

    pltpu.async_copy(dur_hbm.at[b], dur_ref, sem).wait()

    def cs_body(i, carry):
        for h in range(2):
            v = dur_ref[pl.ds(i * 32 + h * 16, 16)]
            s = lane_cumsum(v) + carry
            csum_ref[pl.ds(i * 32 + h * 16, 16)] = s
            carry = s[15]
        return carry

    total = lax.fori_loop(0, SV // 2, cs_body, jnp.int32(0))

    @pl.when(q == 0)
    def _():
        lens_v[...] = jnp.where(iota16 == 0, total, 0)
        pltpu.async_copy(lens_v, len_hbm.at[b], sem)

    def chunk_cut(l):
        tc0 = (q + l * TPB) * CHUNK
        return tc0, jnp.clip(total - tc0, 0, CHUNK)

    def stage1(l, p):
        tc0, cut = chunk_cut(l)

        @pl.when(cut > 0)
        def _():
            def idx_body(v, carry):
                ja, jb = carry
                t = tc0 + v * 16 + iota16
                j = jnp.zeros((16,), jnp.int32)
                step = 1024
                for _u in range(11):
                    probe = plsc.load_gather(csum_ref, [j + (step - 1)])
                    j = jnp.where(probe <= t, j + step, j)
                    step //= 2
                idx4[p, pl.ds(v * 16, 16)] = (
                    src_base + jnp.minimum(j, S - 1))
                ja = jnp.where(v == 0, j[0], ja)
                jb = jnp.maximum(jb, lane_max(jnp.where(t < total, j, 0)))
                return ja, jb

            ja, jb = lax.fori_loop(0, CV, idx_body,
                                   (jnp.int32(0), jnp.int32(0)))
            ja = jnp.minimum((ja >> 3) << 3, S - SPAN)
            ja = pl.multiple_of(ja, 8)
            ok = (jb - ja < SPAN).astype(jnp.int32)
            meta4[p, :] = jnp.where(iota16 == 0, ja,
                                    jnp.where(iota16 == 1, ok, 0))

            @pl.when(ok == 1)
            def _():
                pltpu.async_copy(
                    x_hbm.at[pl.ds(src_base + ja, SPAN)], blks[p], gsems[p])

    def stage2(l, p, o):
        tc0, cut = chunk_cut(l)
        dst = out_hbm.at[pl.ds(b * MAXM + tc0, CHUNK)]

        @pl.when(l >= STAGE_LAG)
        def _():
            dst_old = out_hbm.at[
                pl.ds(b * MAXM + tc0 - STAGE_LAG * TPB * CHUNK, CHUNK)]
            pltpu.make_async_copy(bufs[o], dst_old, ssems[o]).wait()

        @pl.when(cut > 0)
        def _():
            meta = meta4[p, :]
            ja = pl.multiple_of(meta[0], 8)
            ok = meta[1]

            @pl.when(ok == 1)
            def _():
                pltpu.make_async_copy(
                    x_hbm.at[pl.ds(src_base + ja, SPAN)],
                    blks[p], gsems[p]).wait()
                base = src_base + ja

                def exp_body(v, _):
                    loc = jnp.clip(idx4[p, pl.ds(v * 16, 16)] - base,
                                   0, SPAN - 1)
                    for lane in range(16):
                        jl = loc[lane]
                        r = v * 16 + lane
                        for col in range(DV):
                            cs = pl.ds(col * 16, 16)
                            bufs[o][r, cs] = blks[p][jl, cs]
                    return 0

                lax.fori_loop(0, CV, exp_body, 0)

            @pl.when(ok == 0)
            def _():
                pltpu.async_copy(
                    x_hbm.at[idx4.at[p]], bufs[o], gsems[p]).wait()

            @pl.when(cut < CHUNK)
            def _():
                def zr(r, _):
                    for col in range(DV):
                        bufs[o][r, pl.ds(col * 16, 16)] = zv
                    return 0

                lax.fori_loop(cut, CHUNK, zr, 0)

            pltpu.async_copy(bufs[o], dst, ssems[o])

        @pl.when(cut <= 0)
        def _():
            pltpu.async_copy(zbuf, dst, ssems[o])

    stage1(jnp.int32(0), 0)
    stage1(jnp.int32(1), 1)

    def z_body(r, _):
        for col in range(DV):
            zbuf[r, pl.ds(col * 16, 16)] = zv
        return 0

    lax.fori_loop(0, CHUNK, z_body, 0)

    def pipe_body(g, _):
        for i in range(NBLK):
            l1 = g * NBLK + i + STAGE_LAG

            @pl.when(l1 < NCHUNK)
            def _(l1=l1, i=i):
                stage1(l1, (i + STAGE_LAG) % NBLK)

            stage2(l1 - STAGE_LAG, i % NBLK, i % NOUT)

        return 0

    lax.fori_loop(0, NCHUNK // NBLK, pipe_body, 0)

    for c in range(NCHUNK - NOUT, NCHUNK):
        o = c % NOUT
        dst = out_hbm.at[pl.ds(b * MAXM + (q + c * TPB) * CHUNK, CHUNK)]
        pltpu.make_async_copy(bufs[o], dst, ssems[o]).wait()

    @pl.when(q == 0)
    def _():
        pltpu.make_async_copy(lens_v, len_hbm.at[b], sem).wait()


def kernel(x, duration, max_mel_len):
    del max_mel_len
    out_flat, lens = _expand(x.reshape(B * S, D), duration)
    return out_flat.reshape(B, MAXM, D), lens[:, 0]


# speedup vs baseline: 1.6717x; 1.6717x over previous
import functools

import jax
import jax.numpy as jnp
from jax import lax
from jax.experimental import pallas as pl
from jax.experimental.pallas import tpu as pltpu
from jax.experimental.pallas import tpu_sc as plsc

B, S, D = 8, 2048, 384
MAXM = 14336
NTILES = 32
TPB = NTILES // B
CHUNK = 64
NCHUNK = MAXM // (CHUNK * TPB)
SV = S // 16
CV = CHUNK // 16
DV = D // 16
NBUF = 4
UNROLL = NBUF
STAGE_LAG = 2


@functools.partial(
    pl.kernel,
    out_type=(
        jax.ShapeDtypeStruct((B * MAXM, D), jnp.float32),
        jax.ShapeDtypeStruct((B, 16), jnp.int32),
    ),
    mesh=plsc.VectorSubcoreMesh(core_axis_name="c", subcore_axis_name="s"),
    compiler_params=pltpu.CompilerParams(needs_layout_passes=False),
    scratch_types=[
        pltpu.VMEM((S,), jnp.int32),
        pltpu.VMEM((S,), jnp.int32),
        pltpu.VMEM((NBUF, CHUNK), jnp.int32),
        pltpu.VMEM((CHUNK, D), jnp.float32),
        pltpu.VMEM((CHUNK, D), jnp.float32),
        pltpu.VMEM((CHUNK, D), jnp.float32),
        pltpu.VMEM((CHUNK, D), jnp.float32),
        pltpu.VMEM((CHUNK, D), jnp.float32),
        pltpu.VMEM((16,), jnp.int32),
        pltpu.SemaphoreType.DMA,
        pltpu.SemaphoreType.DMA,
        pltpu.SemaphoreType.DMA,
        pltpu.SemaphoreType.DMA,
        pltpu.SemaphoreType.DMA,
        pltpu.SemaphoreType.DMA,
        pltpu.SemaphoreType.DMA,
        pltpu.SemaphoreType.DMA,
        pltpu.SemaphoreType.DMA,
    ],
)
def _expand(x_hbm, dur_hbm, out_hbm, len_hbm,
            dur_ref, csum_ref, idx4, buf0, buf1, buf2, buf3, zbuf, lens_v,
            sem, gsem0, gsem1, gsem2, gsem3, ssem0, ssem1, ssem2, ssem3):
    cid = lax.axis_index("c")
    sid = lax.axis_index("s")
    wid = cid * 16 + sid
    b = wid // TPB
    q = wid % TPB
    src_base = b * S

    bufs = (buf0, buf1, buf2, buf3)
    gsems = (gsem0, gsem1, gsem2, gsem3)
    ssems = (ssem0, ssem1, ssem2, ssem3)

    iota16 = lax.broadcasted_iota(jnp.int32, (16,), 0)
    zv = jnp.zeros((16,), jnp.float32)

    gather_dnums = lax.GatherDimensionNumbers(
        offset_dims=(), collapsed_slice_dims=(0,), start_index_map=(0,))

    def lane_permute(v, idx):
        return lax.gather(v, idx[:, None], gather_dnums, slice_sizes=(1,),
                          mode=lax.GatherScatterMode.PROMISE_IN_BOUNDS)

    def lane_cumsum(v):
        s = v
        for k in (1, 2, 4, 8):
            sh = lane_permute(s, jnp.maximum(iota16 - k, 0))
            s = s + jnp.where(iota16 >= k, sh, 0)
        return s

    pltpu.async_copy(dur_hbm.at[b], dur_ref, sem).wait()

    def cs_body(i, carry):
        for h in range(2):
            v = dur_ref[pl.ds(i * 32 + h * 16, 16)]
            s = lane_cumsum(v) + carry
            csum_ref[pl.ds(i * 32 + h * 16, 16)] = s
            carry = s[15]
        return carry

    total = lax.fori_loop(0, SV // 2, cs_body, jnp.int32(0))

    @pl.when(q == 0)
    def _():
        lens_v[...] = jnp.where(iota16 == 0, total, 0)
        pltpu.async_copy(lens_v, len_hbm.at[b], sem)

    def chunk_cut(l):
        tc0 = (q + l * TPB) * CHUNK
        return tc0, jnp.clip(total - tc0, 0, CHUNK)

    def stage1(l, p, first=False):
        tc0, cut = chunk_cut(l)

        @pl.when(cut > 0)
        def _():
            def idx_body(v, _):
                t = tc0 + v * 16 + iota16
                j = jnp.zeros((16,), jnp.int32)
                step = 1024
                for _u in range(11):
                    probe = plsc.load_gather(csum_ref, [j + (step - 1)])
                    j = jnp.where(probe <= t, j + step, j)
                    step //= 2
                idx4[p, pl.ds(v * 16, 16)] = (
                    src_base + jnp.minimum(j, S - 1))
                return 0

            lax.fori_loop(0, CV, idx_body, 0)

        if not first:
            @pl.when(l >= NBUF)
            def _():
                dst_old = out_hbm.at[
                    pl.ds(b * MAXM + tc0 - NBUF * TPB * CHUNK, CHUNK)]
                pltpu.make_async_copy(bufs[p], dst_old, ssems[p]).wait()

        @pl.when(cut > 0)
        def _():
            h = CHUNK // 2
            pltpu.async_copy(x_hbm.at[idx4.at[p, pl.ds(0, h)]],
                             bufs[p].at[pl.ds(0, h)], gsems[p])
            pltpu.async_copy(x_hbm.at[idx4.at[p, pl.ds(h, h)]],
                             bufs[p].at[pl.ds(h, h)], gsems[p])

    def stage2(l, p):
        tc0, cut = chunk_cut(l)
        dst = out_hbm.at[pl.ds(b * MAXM + tc0, CHUNK)]

        @pl.when(cut > 0)
        def _():
            pltpu.make_async_copy(
                x_hbm.at[idx4.at[p]], bufs[p], gsems[p]).wait()

            @pl.when(cut < CHUNK)
            def _():
                def zr(r, _):
                    for col in range(DV):
                        bufs[p][r, pl.ds(col * 16, 16)] = zv
                    return 0

                lax.fori_loop(cut, CHUNK, zr, 0)

            pltpu.async_copy(bufs[p], dst, ssems[p])

        @pl.when(cut <= 0)
        def _():
            pltpu.async_copy(zbuf, dst, ssems[p])

    stage1(jnp.int32(0), 0, first=True)
    stage1(jnp.int32(1), 1, first=True)

    def z_body(r, _):
        for col in range(DV):
            zbuf[r, pl.ds(col * 16, 16)] = zv
        return 0

    lax.fori_loop(0, CHUNK, z_body, 0)

    def pipe_body(g, _):
        for i in range(UNROLL):
            l1 = g * UNROLL + i + STAGE_LAG

            @pl.when(l1 < NCHUNK)
            def _(l1=l1, i=i):
                stage1(l1, (i + STAGE_LAG) % NBUF)

            l2 = l1 - STAGE_LAG

            @pl.when(l2 < NCHUNK)
            def _(l2=l2, m=i % NBUF):
                stage2(l2, m)

        return 0

    niter = (NCHUNK + UNROLL - 1) // UNROLL
    lax.fori_loop(0, niter, pipe_body, 0)

    for c in range(NCHUNK - NBUF, NCHUNK):
        p = c % NBUF
        dst = out_hbm.at[pl.ds(b * MAXM + (q + c * TPB) * CHUNK, CHUNK)]
        pltpu.make_async_copy(bufs[p], dst, ssems[p]).wait()

    @pl.when(q == 0)
    def _():
        pltpu.make_async_copy(lens_v, len_hbm.at[b], sem).wait()


def kernel(x, duration, max_mel_len):
    del max_mel_len
    out_flat, lens = _expand(x.reshape(B * S, D), duration)
    return out_flat.reshape(B, MAXM, D), lens[:, 0]


# speedup vs baseline: 1.6808x; 1.0055x over previous
import functools

import jax
import jax.numpy as jnp
from jax import lax
from jax.experimental import pallas as pl
from jax.experimental.pallas import tpu as pltpu
from jax.experimental.pallas import tpu_sc as plsc

B, S, D = 8, 2048, 384
MAXM = 14336
NTILES = 32
TPB = NTILES // B
CHUNK = 64
NCHUNK = MAXM // (CHUNK * TPB)
SV = S // 16
CV = CHUNK // 16
DV = D // 16
NBUF = 4
UNROLL = NBUF
STAGE_LAG = 3


@functools.partial(
    pl.kernel,
    out_type=(
        jax.ShapeDtypeStruct((B * MAXM, D), jnp.float32),
        jax.ShapeDtypeStruct((B, 16), jnp.int32),
    ),
    mesh=plsc.VectorSubcoreMesh(core_axis_name="c", subcore_axis_name="s"),
    compiler_params=pltpu.CompilerParams(needs_layout_passes=False),
    scratch_types=[
        pltpu.VMEM((S,), jnp.int32),
        pltpu.VMEM((S,), jnp.int32),
        pltpu.VMEM((NBUF, CHUNK), jnp.int32),
        pltpu.VMEM((CHUNK, D), jnp.float32),
        pltpu.VMEM((CHUNK, D), jnp.float32),
        pltpu.VMEM((CHUNK, D), jnp.float32),
        pltpu.VMEM((CHUNK, D), jnp.float32),
        pltpu.VMEM((CHUNK, D), jnp.float32),
        pltpu.VMEM((16,), jnp.int32),
        pltpu.SemaphoreType.DMA,
        pltpu.SemaphoreType.DMA,
        pltpu.SemaphoreType.DMA,
        pltpu.SemaphoreType.DMA,
        pltpu.SemaphoreType.DMA,
        pltpu.SemaphoreType.DMA,
        pltpu.SemaphoreType.DMA,
        pltpu.SemaphoreType.DMA,
        pltpu.SemaphoreType.DMA,
    ],
)
def _expand(x_hbm, dur_hbm, out_hbm, len_hbm,
            dur_ref, csum_ref, idx4, buf0, buf1, buf2, buf3, zbuf, lens_v,
            sem, gsem0, gsem1, gsem2, gsem3, ssem0, ssem1, ssem2, ssem3):
    cid = lax.axis_index("c")
    sid = lax.axis_index("s")
    wid = cid * 16 + sid
    b = wid // TPB
    q = wid % TPB
    src_base = b * S

    bufs = (buf0, buf1, buf2, buf3)
    gsems = (gsem0, gsem1, gsem2, gsem3)
    ssems = (ssem0, ssem1, ssem2, ssem3)

    iota16 = lax.broadcasted_iota(jnp.int32, (16,), 0)
    zv = jnp.zeros((16,), jnp.float32)

    gather_dnums = lax.GatherDimensionNumbers(
        offset_dims=(), collapsed_slice_dims=(0,), start_index_map=(0,))

    def lane_permute(v, idx):
        return lax.gather(v, idx[:, None], gather_dnums, slice_sizes=(1,),
                          mode=lax.GatherScatterMode.PROMISE_IN_BOUNDS)

    def lane_cumsum(v):
        s = v
        for k in (1, 2, 4, 8):
            sh = lane_permute(s, jnp.maximum(iota16 - k, 0))
            s = s + jnp.where(iota16 >= k, sh, 0)
        return s

    pltpu.async_copy(dur_hbm.at[b], dur_ref, sem).wait()

    def cs_body(i, carry):
        for h in range(2):
            v = dur_ref[pl.ds(i * 32 + h * 16, 16)]
            s = lane_cumsum(v) + carry
            csum_ref[pl.ds(i * 32 + h * 16, 16)] = s
            carry = s[15]
        return carry

    total = lax.fori_loop(0, SV // 2, cs_body, jnp.int32(0))

    @pl.when(q == 0)
    def _():
        lens_v[...] = jnp.where(iota16 == 0, total, 0)
        pltpu.async_copy(lens_v, len_hbm.at[b], sem)

    def chunk_cut(l):
        tc0 = (q + l * TPB) * CHUNK
        return tc0, jnp.clip(total - tc0, 0, CHUNK)

    def stage1(l, p, first=False):
        tc0, cut = chunk_cut(l)

        @pl.when(cut > 0)
        def _():
            def idx_body(v, _):
                t = tc0 + v * 16 + iota16
                j = jnp.zeros((16,), jnp.int32)
                step = 1024
                for _u in range(11):
                    probe = plsc.load_gather(csum_ref, [j + (step - 1)])
                    j = jnp.where(probe <= t, j + step, j)
                    step //= 2
                idx4[p, pl.ds(v * 16, 16)] = (
                    src_base + jnp.minimum(j, S - 1))
                return 0

            lax.fori_loop(0, CV, idx_body, 0)

        if not first:
            @pl.when(l >= NBUF)
            def _():
                dst_old = out_hbm.at[
                    pl.ds(b * MAXM + tc0 - NBUF * TPB * CHUNK, CHUNK)]
                pltpu.make_async_copy(bufs[p], dst_old, ssems[p]).wait()

        @pl.when(cut > 0)
        def _():
            h = CHUNK // 2
            pltpu.async_copy(x_hbm.at[idx4.at[p, pl.ds(0, h)]],
                             bufs[p].at[pl.ds(0, h)], gsems[p])
            pltpu.async_copy(x_hbm.at[idx4.at[p, pl.ds(h, h)]],
                             bufs[p].at[pl.ds(h, h)], gsems[p])

    def stage2(l, p):
        tc0, cut = chunk_cut(l)
        dst = out_hbm.at[pl.ds(b * MAXM + tc0, CHUNK)]

        @pl.when(cut > 0)
        def _():
            pltpu.make_async_copy(
                x_hbm.at[idx4.at[p]], bufs[p], gsems[p]).wait()

            @pl.when(cut < CHUNK)
            def _():
                def zr(r, _):
                    for col in range(DV):
                        bufs[p][r, pl.ds(col * 16, 16)] = zv
                    return 0

                lax.fori_loop(cut, CHUNK, zr, 0)

            pltpu.async_copy(bufs[p], dst, ssems[p])

        @pl.when(cut <= 0)
        def _():
            pltpu.async_copy(zbuf, dst, ssems[p])

    stage1(jnp.int32(0), 0, first=True)
    stage1(jnp.int32(1), 1, first=True)
    stage1(jnp.int32(2), 2, first=True)

    def z_body(r, _):
        for col in range(DV):
            zbuf[r, pl.ds(col * 16, 16)] = zv
        return 0

    lax.fori_loop(0, CHUNK, z_body, 0)

    def pipe_body(g, _):
        for i in range(UNROLL):
            l1 = g * UNROLL + i + STAGE_LAG

            @pl.when(l1 < NCHUNK)
            def _(l1=l1, i=i):
                stage1(l1, (i + STAGE_LAG) % NBUF)

            l2 = l1 - STAGE_LAG

            @pl.when(l2 < NCHUNK)
            def _(l2=l2, m=i % NBUF):
                stage2(l2, m)

        return 0

    niter = (NCHUNK + UNROLL - 1) // UNROLL
    lax.fori_loop(0, niter, pipe_body, 0)

    for c in range(NCHUNK - NBUF, NCHUNK):
        p = c % NBUF
        dst = out_hbm.at[pl.ds(b * MAXM + (q + c * TPB) * CHUNK, CHUNK)]
        pltpu.make_async_copy(bufs[p], dst, ssems[p]).wait()

    @pl.when(q == 0)
    def _():
        pltpu.make_async_copy(lens_v, len_hbm.at[b], sem).wait()


def kernel(x, duration, max_mel_len):
    del max_mel_len
    out_flat, lens = _expand(x.reshape(B * S, D), duration)
    return out_flat.reshape(B, MAXM, D), lens[:, 0]
